# Initial kernel scaffold; baseline (speedup 1.0000x reference)
import jax
import jax.numpy as jnp
from jax.experimental import pallas as pl


def kernel(stock_feat, bank_feat, industry_feat, edge_index_ss, edge_index_sb, edge_index_si, edge_index_bs, edge_index_is, edge_index_ii, params):
    raise NotImplementedError("write your pallas kernel here")


# speedup vs baseline: 88.1742x; 88.1742x over previous
import functools

import jax
import jax.numpy as jnp
import numpy as np
from jax import lax
from jax.experimental import pallas as pl
from jax.experimental.pallas import tpu as pltpu
from jax.experimental.pallas import tpu_sc as plsc

HID = 128
NH = 4
DH = HID // NH
LW = HID + 16

NC, NS = 2, 16
NW = NC * NS
WIN = 128

BB = 400

A1, OFF_SB1, OFF_SI1, PAD1 = 12288, 10000, 11200, 11600
A2, OFF_IS2, PAD2 = 2048, 1200, 1600
A3, PAD3 = 1024, 400

_f32 = jnp.float32


def _round_up(x, m):
    return (x + m - 1) // m * m


_G = np.kron(np.eye(NH), np.ones((DH, DH))).astype(np.float32)
_P = np.kron(np.eye(NH), np.ones((DH, NH)) / DH).astype(np.float32)
_Q = np.kron(np.eye(NH), np.ones((NH, DH)) / NH).astype(np.float32)
_H = (np.tile(np.eye(DH), (NH, 1)) / NH).astype(np.float32)


def _prep_rga(p):
    ws, bs = p['src_proj']
    wm, bm = p['msg_fc']
    wa = p['attn_fc'][0][0]
    wmerge, bmerge = p['merge_fc']
    wo, bo = p['out_fc']
    g, b = p['norm']
    eye = jnp.eye(NH, dtype=_f32)
    return dict(
        wsT=ws.T,
        bs=bs[None, :],
        wmbig=jnp.kron(eye, wm.T),
        bmrep=jnp.tile(bm, NH)[None, :],
        warep=jnp.tile(wa[:DH], NH)[None, :],
        m1=_H @ wmerge.T,
        bmerge=bmerge[None, :],
        woaT=wo[:, :HID].T,
        wobT=wo[:, HID:].T,
        bout=bo[None, :],
        g=g[None, :],
        b=b[None, :],
    )


def _enc_body(x, w1T, b1, w2T, b2, o):
    h = jnp.maximum(x[...] @ w1T[...] + b1[...], 0.0)
    o[...] = h @ w2T[...] + b2[...]


def _enc(x, p, nblocks):
    w1, b1 = p[0]
    w2, b2 = p[1]
    n, f = x.shape
    full = lambda a: pl.BlockSpec(a.shape, lambda i: (0,) * a.ndim)
    return pl.pallas_call(
        _enc_body,
        grid=(nblocks,),
        in_specs=[
            pl.BlockSpec((n // nblocks, f), lambda i: (i, 0)),
            full(w1.T), full(b1[None, :]), full(w2.T), full(b2[None, :]),
        ],
        out_specs=pl.BlockSpec((n // nblocks, HID), lambda i: (i, 0)),
        out_shape=jax.ShapeDtypeStruct((n, HID), _f32),
    )(x, w1.T, b1[None, :], w2.T, b2[None, :])


def _tbl_body(x, wsT, bs, wmbig, bmrep, warep, G, P, o):
    sf = x[...] @ wsT[...] + bs[...]
    t = jnp.tanh(sf)
    ab = (t * warep[...]) @ G[...]
    scale = jnp.exp(ab)
    wmsg = (sf @ wmbig[...] + bmrep[...]) * scale
    den16 = scale @ P[...]
    o[...] = jnp.concatenate([wmsg, den16], axis=1)


def _tbl(x, w, nblocks):
    n = x.shape[0]
    full = lambda a: pl.BlockSpec(a.shape, lambda i: (0,) * a.ndim)
    args = (x, w['wsT'], w['bs'], w['wmbig'], w['bmrep'], w['warep'], _G, _P)
    return pl.pallas_call(
        _tbl_body,
        grid=(nblocks,),
        in_specs=[pl.BlockSpec((n // nblocks, HID), lambda i: (i, 0))]
        + [full(a) for a in args[1:]],
        out_specs=pl.BlockSpec((n // nblocks, LW), lambda i: (i, 0)),
        out_shape=jax.ShapeDtypeStruct((n, LW), _f32),
    )(*args)


def _epi_math(acc, x, w, q):
    den_b = acc[:, HID:] @ q + 1e-12
    r = acc[:, :HID] / den_b
    merge = r @ w['m1'] + w['bmerge'][...]
    upd = x @ w['woaT'][...] + merge @ w['wobT'][...] + w['bout'][...]
    y = x + upd
    mu = jnp.mean(y, -1, keepdims=True)
    var = jnp.mean((y - mu) ** 2, -1, keepdims=True)
    return (y - mu) * lax.rsqrt(var + 1e-5) * w['g'][...] + w['b'][...]


_RGA_KEYS = ('m1', 'bmerge', 'woaT', 'wobT', 'bout', 'g', 'b')


def _epi_body(a0, a1, x, q, m1, bmerge, woaT, wobT, bout, g, b, o):
    w = dict(m1=m1[...], bmerge=bmerge, woaT=woaT, wobT=wobT, bout=bout, g=g, b=b)
    o[...] = _epi_math(a0[0] + a1[0], x[...], w, q[...])


def _epi(x, acc, w, nblocks):
    n = x.shape[0]
    bb = n // nblocks
    full = lambda a: pl.BlockSpec(a.shape, lambda i: (0,) * a.ndim)
    wargs = tuple(w[k] for k in _RGA_KEYS)
    return pl.pallas_call(
        _epi_body,
        grid=(nblocks,),
        in_specs=[
            pl.BlockSpec((1, bb, LW), lambda i: (0, i, 0)),
            pl.BlockSpec((1, bb, LW), lambda i: (1, i, 0)),
            pl.BlockSpec((bb, HID), lambda i: (i, 0)),
            full(_Q),
        ] + [full(a) for a in wargs],
        out_specs=pl.BlockSpec((bb, HID), lambda i: (i, 0)),
        out_shape=jax.ShapeDtypeStruct((n, HID), _f32),
    )(acc, acc, x, _Q, *wargs)


def _mega_body(x, ss0, ss1, sb0, sb1, si0, si1, q,
               *flat, nsb_blocks, nsi_blocks):
    wss = dict(zip(_RGA_KEYS, flat[0:7]))
    wsb = dict(zip(_RGA_KEYS, flat[7:14]))
    wsi = dict(zip(_RGA_KEYS, flat[14:21]))
    wpT, bp, wsrep, woT, bo, gm, bm = flat[21:28]
    o = flat[28]
    i = pl.program_id(0)
    msb = jnp.where(i < nsb_blocks, 1.0, 0.0).astype(_f32)
    msi = jnp.where(i < nsi_blocks, 1.0, 0.0).astype(_f32)
    xv = x[...]
    qv = q[...]
    p0 = xv
    p1 = _epi_math(ss0[0] + ss1[0], xv, {k: v[...] if k == 'm1' else v for k, v in wss.items()}, qv)
    p2 = _epi_math((sb0[0] + sb1[0]) * msb, xv, {k: v[...] if k == 'm1' else v for k, v in wsb.items()}, qv)
    p3 = _epi_math((si0[0] + si1[0]) * msi, xv, {k: v[...] if k == 'm1' else v for k, v in wsi.items()}, qv)
    paths = (p0, p1, p2, p3)
    scores = [jnp.sum(jnp.tanh(pp @ wpT[...] + bp[...]) * wsrep[...], -1, keepdims=True)
              for pp in paths]
    m = jnp.maximum(jnp.maximum(scores[0], scores[1]),
                    jnp.maximum(scores[2], scores[3]))
    es = [jnp.exp(s - m) for s in scores]
    den = es[0] + es[1] + es[2] + es[3]
    mix = sum(e * pp for e, pp in zip(es, paths)) / den
    y = mix @ woT[...] + bo[...]
    mu = jnp.mean(y, -1, keepdims=True)
    var = jnp.mean((y - mu) ** 2, -1, keepdims=True)
    o[...] = (y - mu) * lax.rsqrt(var + 1e-5) * gm[...] + bm[...]


def _mega(x, acc, wss, wsb, wsi, meta):
    n = x.shape[0]
    nblocks = n // BB
    sb0 = OFF_SB1 // BB
    si0 = OFF_SI1 // BB
    nsb = (OFF_SI1 - OFF_SB1) // BB
    nsi = (PAD1 - OFF_SI1) // BB
    full = lambda a: pl.BlockSpec(a.shape, lambda i: (0,) * a.ndim)
    wsm = meta['path_fc'][0].T
    bpm = meta['path_fc'][1][None, :]
    wsrep = meta['score_fc'][0][None, :]
    wom = meta['out_fc'][0].T
    bom = meta['out_fc'][1][None, :]
    gm = meta['norm'][0][None, :]
    bm = meta['norm'][1][None, :]
    wargs = (tuple(wss[k] for k in _RGA_KEYS) + tuple(wsb[k] for k in _RGA_KEYS)
             + tuple(wsi[k] for k in _RGA_KEYS)
             + (wsm, bpm, wsrep, wom, bom, gm, bm))
    return pl.pallas_call(
        functools.partial(_mega_body, nsb_blocks=nsb, nsi_blocks=nsi),
        grid=(nblocks,),
        in_specs=[
            pl.BlockSpec((BB, HID), lambda i: (i, 0)),
            pl.BlockSpec((1, BB, LW), lambda i: (0, i, 0)),
            pl.BlockSpec((1, BB, LW), lambda i: (1, i, 0)),
            pl.BlockSpec((1, BB, LW), lambda i: (0, sb0 + jnp.minimum(i, nsb - 1), 0)),
            pl.BlockSpec((1, BB, LW), lambda i: (1, sb0 + jnp.minimum(i, nsb - 1), 0)),
            pl.BlockSpec((1, BB, LW), lambda i: (0, si0 + jnp.minimum(i, nsi - 1), 0)),
            pl.BlockSpec((1, BB, LW), lambda i: (1, si0 + jnp.minimum(i, nsi - 1), 0)),
            full(_Q),
        ] + [full(a) for a in wargs],
        out_specs=pl.BlockSpec((BB, HID), lambda i: (i, 0)),
        out_shape=jax.ShapeDtypeStruct((n, HID), _f32),
    )(x, acc, acc, acc, acc, acc, acc, _Q, *wargs)


@functools.cache
def _make_scagg(A, E):
    C = E // NW
    nwin = C // WIN
    rz = A // NS
    assert rz % 8 == 0
    zc = next(c for c in (128, 64, 32, 16, 8) if rz % c == 0)
    mesh = plsc.VectorSubcoreMesh(core_axis_name="c", subcore_axis_name="s",
                                  num_cores=NC, num_subcores=NS)

    @functools.partial(
        pl.kernel,
        out_type=jax.ShapeDtypeStruct((NC, A, LW), _f32),
        mesh=mesh,
        compiler_params=pltpu.CompilerParams(use_tc_tiling_on_sc=False),
        scratch_types=[
            pltpu.VMEM((WIN,), jnp.int32),
            pltpu.VMEM((WIN,), jnp.int32),
            pltpu.VMEM((WIN, LW), _f32),
            pltpu.VMEM_SHARED((A, LW), _f32),
            pltpu.SemaphoreType.DMA,
        ],
    )
    def scagg(table, srcs, dsts, zrows, out, idx_s, idx_d, rows, accum, sem):
        c = lax.axis_index("c")
        s = lax.axis_index("s")
        wid = s * NC + c
        pltpu.sync_copy(zrows.at[pl.ds(0, zc)], rows.at[pl.ds(0, zc)])
        for j in range(rz // zc):
            pltpu.sync_copy(rows.at[pl.ds(0, zc)], accum.at[pl.ds(s * rz + j * zc, zc)])
        plsc.subcore_barrier()
        base = wid * C

        def win(i, carry):
            b = base + i * WIN
            pltpu.sync_copy(srcs.at[pl.ds(b, WIN)], idx_s)
            pltpu.sync_copy(dsts.at[pl.ds(b, WIN)], idx_d)
            pltpu.async_copy(table.at[idx_s], rows, sem).wait()
            pltpu.sync_copy(rows, accum.at[idx_d], add=True)
            return carry

        lax.fori_loop(0, nwin, win, 0)
        plsc.subcore_barrier()
        pltpu.sync_copy(accum.at[pl.ds(s * rz, rz)], out.at[c, pl.ds(s * rz, rz)])

    return scagg


def _scagg(table, src, dst, A, E):
    zrows = jnp.zeros((128, LW), _f32)
    return _make_scagg(A, E)(table, src, dst, zrows)


def _cat_edges(parts, pad_base, total):
    srcs = [e[0].astype(jnp.int32) + off for e, off in parts]
    dsts = [e[1].astype(jnp.int32) + off for e, off in parts]
    ne = sum(s.shape[0] for s in srcs)
    npad = total - ne
    pad = pad_base + (jnp.arange(npad, dtype=jnp.int32) % 8)
    return jnp.concatenate(srcs + [pad]), jnp.concatenate(dsts + [pad])


def kernel(stock_feat, bank_feat, industry_feat, edge_index_ss, edge_index_sb,
           edge_index_si, edge_index_bs, edge_index_is, edge_index_ii, params):
    stock_h = _enc(stock_feat, params['stock_enc'], 25)
    bank_h = _enc(jnp.pad(bank_feat, ((0, 200), (0, 0))), params['bank_enc'], 3)
    ind_h = _enc(jnp.pad(industry_feat, ((0, 300), (0, 0))), params['industry_enc'], 1)

    e1 = _round_up(edge_index_ss.shape[1] + edge_index_sb.shape[1]
                   + edge_index_si.shape[1], NW * WIN)
    src1, dst1 = _cat_edges([(edge_index_ss, 0), (edge_index_sb, OFF_SB1),
                             (edge_index_si, OFF_SI1)], PAD1, e1)
    e2 = _round_up(edge_index_bs.shape[1] + edge_index_is.shape[1], NW * WIN)
    src2, dst2 = _cat_edges([(edge_index_bs, 0), (edge_index_is, OFF_IS2)], PAD2, e2)
    e3 = _round_up(edge_index_ii.shape[1], NW * WIN)
    src3, dst3 = _cat_edges([(edge_index_ii, 0)], PAD3, e3)

    zpad1 = jnp.zeros((A1 - PAD1, LW), _f32)
    zpad2 = jnp.zeros((A2 - PAD2, LW), _f32)
    zpad3 = jnp.zeros((A3 - PAD3, LW), _f32)

    for lp in params['layers']:
        wss, wsb, wsi = _prep_rga(lp['SS']), _prep_rga(lp['SB']), _prep_rga(lp['SI'])
        wbs, wis, wii = _prep_rga(lp['BS']), _prep_rga(lp['IS']), _prep_rga(lp['II'])

        t1 = jnp.concatenate([_tbl(stock_h, wss, 25), _tbl(bank_h, wsb, 3),
                              _tbl(ind_h, wsi, 1), zpad1])
        acc1 = _scagg(t1, src1, dst1, A1, e1)
        stock_h = _mega(stock_h, acc1, wss, wsb, wsi, params['meta'])

        t2 = jnp.concatenate([_tbl(stock_h[:1200], wbs, 3),
                              _tbl(stock_h[:400], wis, 1), zpad2])
        acc2 = _scagg(t2, src2, dst2, A2, e2)
        bank_h = _epi(bank_h, acc2[:, :OFF_IS2], wbs, 3)
        ind_h = _epi(ind_h, acc2[:, OFF_IS2:PAD2], wis, 1)

        t3 = jnp.concatenate([_tbl(ind_h, wii, 1), zpad3])
        acc3 = _scagg(t3, src3, dst3, A3, e3)
        ind_h = _epi(ind_h, acc3[:, :PAD3], wii, 1)

    return stock_h


# speedup vs baseline: 105.8358x; 1.2003x over previous
import functools

import jax
import jax.numpy as jnp
import numpy as np
from jax import lax
from jax.experimental import pallas as pl
from jax.experimental.pallas import tpu as pltpu
from jax.experimental.pallas import tpu_sc as plsc

HID = 128
NH = 4
DH = HID // NH
LW = HID + 16

NC, NS = 2, 16
NW = NC * NS
WIN = 128

BB = 400

A1, OFF_SB1, OFF_SI1, PAD1 = 12288, 10000, 11200, 11600
A2, OFF_IS2, PAD2 = 2048, 1200, 1600
A3, PAD3 = 1024, 400

_f32 = jnp.float32


def _round_up(x, m):
    return (x + m - 1) // m * m


_G = np.kron(np.eye(NH), np.ones((DH, DH))).astype(np.float32)
_P = np.kron(np.eye(NH), np.ones((DH, NH)) / DH).astype(np.float32)
_Q = np.kron(np.eye(NH), np.ones((NH, DH)) / NH).astype(np.float32)
_H = (np.tile(np.eye(DH), (NH, 1)) / NH).astype(np.float32)


def _prep_rga(p):
    ws, bs = p['src_proj']
    wm, bm = p['msg_fc']
    wa = p['attn_fc'][0][0]
    wmerge, bmerge = p['merge_fc']
    wo, bo = p['out_fc']
    g, b = p['norm']
    eye = jnp.eye(NH, dtype=_f32)
    return dict(
        wsT=ws.T,
        bs=bs[None, :],
        wmbig=jnp.kron(eye, wm.T),
        bmrep=jnp.tile(bm, NH)[None, :],
        warep=jnp.tile(wa[:DH], NH)[None, :],
        m1=_H @ wmerge.T,
        bmerge=bmerge[None, :],
        woaT=wo[:, :HID].T,
        wobT=wo[:, HID:].T,
        bout=bo[None, :],
        g=g[None, :],
        b=b[None, :],
    )


def _enc_body(x, w1T, b1, w2T, b2, o):
    h = jnp.maximum(x[...] @ w1T[...] + b1[...], 0.0)
    o[...] = h @ w2T[...] + b2[...]


def _enc(x, p, nblocks):
    w1, b1 = p[0]
    w2, b2 = p[1]
    n, f = x.shape
    full = lambda a: pl.BlockSpec(a.shape, lambda i: (0,) * a.ndim)
    return pl.pallas_call(
        _enc_body,
        grid=(nblocks,),
        in_specs=[
            pl.BlockSpec((n // nblocks, f), lambda i: (i, 0)),
            full(w1.T), full(b1[None, :]), full(w2.T), full(b2[None, :]),
        ],
        out_specs=pl.BlockSpec((n // nblocks, HID), lambda i: (i, 0)),
        out_shape=jax.ShapeDtypeStruct((n, HID), _f32),
    )(x, w1.T, b1[None, :], w2.T, b2[None, :])


def _tbl_body(x, wsT, bs, wmbig, bmrep, warep, G, P, o):
    sf = x[...] @ wsT[...] + bs[...]
    t = jnp.tanh(sf)
    ab = (t * warep[...]) @ G[...]
    scale = jnp.exp(ab)
    wmsg = (sf @ wmbig[...] + bmrep[...]) * scale
    den16 = scale @ P[...]
    o[...] = jnp.concatenate([wmsg, den16], axis=1)


def _tbl(x, w, nblocks):
    n = x.shape[0]
    full = lambda a: pl.BlockSpec(a.shape, lambda i: (0,) * a.ndim)
    args = (x, w['wsT'], w['bs'], w['wmbig'], w['bmrep'], w['warep'], _G, _P)
    return pl.pallas_call(
        _tbl_body,
        grid=(nblocks,),
        in_specs=[pl.BlockSpec((n // nblocks, HID), lambda i: (i, 0))]
        + [full(a) for a in args[1:]],
        out_specs=pl.BlockSpec((n // nblocks, LW), lambda i: (i, 0)),
        out_shape=jax.ShapeDtypeStruct((n, LW), _f32),
    )(*args)


def _epi_math(acc, x, w, q):
    den_b = acc[:, HID:] @ q + 1e-12
    r = acc[:, :HID] / den_b
    merge = r @ w['m1'] + w['bmerge'][...]
    upd = x @ w['woaT'][...] + merge @ w['wobT'][...] + w['bout'][...]
    y = x + upd
    mu = jnp.mean(y, -1, keepdims=True)
    var = jnp.mean((y - mu) ** 2, -1, keepdims=True)
    return (y - mu) * lax.rsqrt(var + 1e-5) * w['g'][...] + w['b'][...]


_RGA_KEYS = ('m1', 'bmerge', 'woaT', 'wobT', 'bout', 'g', 'b')


def _epi_body(a0, a1, x, q, m1, bmerge, woaT, wobT, bout, g, b, o):
    w = dict(m1=m1[...], bmerge=bmerge, woaT=woaT, wobT=wobT, bout=bout, g=g, b=b)
    o[...] = _epi_math(a0[0] + a1[0], x[...], w, q[...])


def _epi(x, acc, w, nblocks):
    n = x.shape[0]
    bb = n // nblocks
    full = lambda a: pl.BlockSpec(a.shape, lambda i: (0,) * a.ndim)
    wargs = tuple(w[k] for k in _RGA_KEYS)
    return pl.pallas_call(
        _epi_body,
        grid=(nblocks,),
        in_specs=[
            pl.BlockSpec((1, bb, LW), lambda i: (0, i, 0)),
            pl.BlockSpec((1, bb, LW), lambda i: (1, i, 0)),
            pl.BlockSpec((bb, HID), lambda i: (i, 0)),
            full(_Q),
        ] + [full(a) for a in wargs],
        out_specs=pl.BlockSpec((bb, HID), lambda i: (i, 0)),
        out_shape=jax.ShapeDtypeStruct((n, HID), _f32),
    )(acc, acc, x, _Q, *wargs)


def _mega_body(x, ss0, ss1, sb0, sb1, si0, si1, q,
               *flat, nsb_blocks, nsi_blocks):
    wss = dict(zip(_RGA_KEYS, flat[0:7]))
    wsb = dict(zip(_RGA_KEYS, flat[7:14]))
    wsi = dict(zip(_RGA_KEYS, flat[14:21]))
    wpT, bp, wsrep, woT, bo, gm, bm = flat[21:28]
    o = flat[28]
    i = pl.program_id(0)
    msb = jnp.where(i < nsb_blocks, 1.0, 0.0).astype(_f32)
    msi = jnp.where(i < nsi_blocks, 1.0, 0.0).astype(_f32)
    xv = x[...]
    qv = q[...]
    p0 = xv
    p1 = _epi_math(ss0[0] + ss1[0], xv, {k: v[...] if k == 'm1' else v for k, v in wss.items()}, qv)
    p2 = _epi_math((sb0[0] + sb1[0]) * msb, xv, {k: v[...] if k == 'm1' else v for k, v in wsb.items()}, qv)
    p3 = _epi_math((si0[0] + si1[0]) * msi, xv, {k: v[...] if k == 'm1' else v for k, v in wsi.items()}, qv)
    paths = (p0, p1, p2, p3)
    scores = [jnp.sum(jnp.tanh(pp @ wpT[...] + bp[...]) * wsrep[...], -1, keepdims=True)
              for pp in paths]
    m = jnp.maximum(jnp.maximum(scores[0], scores[1]),
                    jnp.maximum(scores[2], scores[3]))
    es = [jnp.exp(s - m) for s in scores]
    den = es[0] + es[1] + es[2] + es[3]
    mix = sum(e * pp for e, pp in zip(es, paths)) / den
    y = mix @ woT[...] + bo[...]
    mu = jnp.mean(y, -1, keepdims=True)
    var = jnp.mean((y - mu) ** 2, -1, keepdims=True)
    o[...] = (y - mu) * lax.rsqrt(var + 1e-5) * gm[...] + bm[...]


def _mega(x, acc, wss, wsb, wsi, meta):
    n = x.shape[0]
    nblocks = n // BB
    sb0 = OFF_SB1 // BB
    si0 = OFF_SI1 // BB
    nsb = (OFF_SI1 - OFF_SB1) // BB
    nsi = (PAD1 - OFF_SI1) // BB
    full = lambda a: pl.BlockSpec(a.shape, lambda i: (0,) * a.ndim)
    wsm = meta['path_fc'][0].T
    bpm = meta['path_fc'][1][None, :]
    wsrep = meta['score_fc'][0][None, :]
    wom = meta['out_fc'][0].T
    bom = meta['out_fc'][1][None, :]
    gm = meta['norm'][0][None, :]
    bm = meta['norm'][1][None, :]
    wargs = (tuple(wss[k] for k in _RGA_KEYS) + tuple(wsb[k] for k in _RGA_KEYS)
             + tuple(wsi[k] for k in _RGA_KEYS)
             + (wsm, bpm, wsrep, wom, bom, gm, bm))
    return pl.pallas_call(
        functools.partial(_mega_body, nsb_blocks=nsb, nsi_blocks=nsi),
        grid=(nblocks,),
        in_specs=[
            pl.BlockSpec((BB, HID), lambda i: (i, 0)),
            pl.BlockSpec((1, BB, LW), lambda i: (0, i, 0)),
            pl.BlockSpec((1, BB, LW), lambda i: (1, i, 0)),
            pl.BlockSpec((1, BB, LW), lambda i: (0, sb0 + jnp.minimum(i, nsb - 1), 0)),
            pl.BlockSpec((1, BB, LW), lambda i: (1, sb0 + jnp.minimum(i, nsb - 1), 0)),
            pl.BlockSpec((1, BB, LW), lambda i: (0, si0 + jnp.minimum(i, nsi - 1), 0)),
            pl.BlockSpec((1, BB, LW), lambda i: (1, si0 + jnp.minimum(i, nsi - 1), 0)),
            full(_Q),
        ] + [full(a) for a in wargs],
        out_specs=pl.BlockSpec((BB, HID), lambda i: (i, 0)),
        out_shape=jax.ShapeDtypeStruct((n, HID), _f32),
    )(x, acc, acc, acc, acc, acc, acc, _Q, *wargs)


def _win_for(A):
    return 64 if A > 4096 else 128


@functools.cache
def _make_scagg(A, E):
    win = _win_for(A)
    C = E // NW
    nwin = C // win
    assert C % win == 0 and nwin % 2 == 0
    niter = nwin // 2
    rz = A // NS
    assert rz % 8 == 0
    zc = next(c for c in (128, 64, 32, 16, 8) if rz % c == 0 and c <= win)
    mesh = plsc.VectorSubcoreMesh(core_axis_name="c", subcore_axis_name="s",
                                  num_cores=NC, num_subcores=NS)

    @functools.partial(
        pl.kernel,
        out_type=jax.ShapeDtypeStruct((NC, A, LW), _f32),
        mesh=mesh,
        compiler_params=pltpu.CompilerParams(use_tc_tiling_on_sc=False),
        scratch_types=[
            pltpu.VMEM((2, 2, win), jnp.int32),
            pltpu.VMEM((win, LW), _f32),
            pltpu.VMEM((win, LW), _f32),
            pltpu.VMEM_SHARED((A, LW), _f32),
            pltpu.SemaphoreType.DMA,
            pltpu.SemaphoreType.DMA,
            pltpu.SemaphoreType.DMA,
            pltpu.SemaphoreType.DMA,
        ],
    )
    def scagg(table, idxs, zrows, out, idxb, rows0, rows1, accum, g0, g1, s0, s1):
        c = lax.axis_index("c")
        s = lax.axis_index("s")
        wid = s * NC + c
        pltpu.sync_copy(zrows.at[pl.ds(0, zc)], rows0.at[pl.ds(0, zc)])
        for j in range(rz // zc):
            pltpu.sync_copy(rows0.at[pl.ds(0, zc)],
                            accum.at[pl.ds(s * rz + j * zc, zc)])
        plsc.subcore_barrier()
        sbase = wid * nwin
        pltpu.sync_copy(idxs.at[sbase], idxb.at[0])
        pltpu.async_copy(table.at[idxb.at[0, 0]], rows0, g0)

        def body(jj, carry):
            @pl.when(jj > 0)
            def _():
                pltpu.make_async_copy(table.at[pl.ds(0, win)], rows1, s1).wait()

            pltpu.sync_copy(idxs.at[sbase + 2 * jj + 1], idxb.at[1])
            pltpu.async_copy(table.at[idxb.at[1, 0]], rows1, g1)
            pltpu.make_async_copy(table.at[pl.ds(0, win)], rows0, g0).wait()
            pltpu.async_copy(rows0, accum.at[idxb.at[0, 1]], s0, add=True)

            @pl.when(jj < niter - 1)
            def _():
                pltpu.make_async_copy(table.at[pl.ds(0, win)], rows0, s0).wait()
                pltpu.sync_copy(idxs.at[sbase + 2 * jj + 2], idxb.at[0])
                pltpu.async_copy(table.at[idxb.at[0, 0]], rows0, g0)

            pltpu.make_async_copy(table.at[pl.ds(0, win)], rows1, g1).wait()
            pltpu.async_copy(rows1, accum.at[idxb.at[1, 1]], s1, add=True)
            return carry

        lax.fori_loop(0, niter, body, 0)
        pltpu.make_async_copy(table.at[pl.ds(0, win)], rows0, s0).wait()
        pltpu.make_async_copy(table.at[pl.ds(0, win)], rows1, s1).wait()
        plsc.subcore_barrier()
        pltpu.sync_copy(accum.at[pl.ds(s * rz, rz)], out.at[c, pl.ds(s * rz, rz)])

    return scagg


def _scagg(table, src, dst, A, E):
    win = _win_for(A)
    idxs = jnp.stack([src.reshape(-1, win), dst.reshape(-1, win)], axis=1)
    zrows = jnp.zeros((128, LW), _f32)
    return _make_scagg(A, E)(table, idxs, zrows)


def _cat_edges(parts, pad_base, total):
    srcs = [e[0].astype(jnp.int32) + off for e, off in parts]
    dsts = [e[1].astype(jnp.int32) + off for e, off in parts]
    ne = sum(s.shape[0] for s in srcs)
    npad = total - ne
    pad = pad_base + (jnp.arange(npad, dtype=jnp.int32) % 8)
    return jnp.concatenate(srcs + [pad]), jnp.concatenate(dsts + [pad])


def kernel(stock_feat, bank_feat, industry_feat, edge_index_ss, edge_index_sb,
           edge_index_si, edge_index_bs, edge_index_is, edge_index_ii, params):
    stock_h = _enc(stock_feat, params['stock_enc'], 25)
    bank_h = _enc(jnp.pad(bank_feat, ((0, 200), (0, 0))), params['bank_enc'], 3)
    ind_h = _enc(jnp.pad(industry_feat, ((0, 300), (0, 0))), params['industry_enc'], 1)

    e1 = _round_up(edge_index_ss.shape[1] + edge_index_sb.shape[1]
                   + edge_index_si.shape[1], NW * 2 * _win_for(A1))
    src1, dst1 = _cat_edges([(edge_index_ss, 0), (edge_index_sb, OFF_SB1),
                             (edge_index_si, OFF_SI1)], PAD1, e1)
    e2 = _round_up(edge_index_bs.shape[1] + edge_index_is.shape[1],
                   NW * 2 * _win_for(A2))
    src2, dst2 = _cat_edges([(edge_index_bs, 0), (edge_index_is, OFF_IS2)], PAD2, e2)
    e3 = _round_up(edge_index_ii.shape[1], NW * 2 * _win_for(A3))
    src3, dst3 = _cat_edges([(edge_index_ii, 0)], PAD3, e3)

    zpad1 = jnp.zeros((A1 - PAD1, LW), _f32)
    zpad2 = jnp.zeros((A2 - PAD2, LW), _f32)
    zpad3 = jnp.zeros((A3 - PAD3, LW), _f32)

    for lp in params['layers']:
        wss, wsb, wsi = _prep_rga(lp['SS']), _prep_rga(lp['SB']), _prep_rga(lp['SI'])
        wbs, wis, wii = _prep_rga(lp['BS']), _prep_rga(lp['IS']), _prep_rga(lp['II'])

        t1 = jnp.concatenate([_tbl(stock_h, wss, 25), _tbl(bank_h, wsb, 3),
                              _tbl(ind_h, wsi, 1), zpad1])
        acc1 = _scagg(t1, src1, dst1, A1, e1)
        stock_h = _mega(stock_h, acc1, wss, wsb, wsi, params['meta'])

        t2 = jnp.concatenate([_tbl(stock_h[:1200], wbs, 3),
                              _tbl(stock_h[:400], wis, 1), zpad2])
        acc2 = _scagg(t2, src2, dst2, A2, e2)
        bank_h = _epi(bank_h, acc2[:, :OFF_IS2], wbs, 3)
        ind_h = _epi(ind_h, acc2[:, OFF_IS2:PAD2], wis, 1)

        t3 = jnp.concatenate([_tbl(ind_h, wii, 1), zpad3])
        acc3 = _scagg(t3, src3, dst3, A3, e3)
        ind_h = _epi(ind_h, acc3[:, :PAD3], wii, 1)

    return stock_h
